# baseline (speedup 1.0000x reference)
import jax
import jax.numpy as jnp
from jax import lax
from jax.experimental import pallas as pl

_N, _M, _D = 10000, 2500, 256

def _mm_body(x_ref, w_ref, o_ref):
    o_ref[...] = lax.dot_general(
        x_ref[...], w_ref[...], (((1,), (1,)), ((), ())),
        preferred_element_type=jnp.float32)

def _linear(X, Wlin):
    return pl.pallas_call(
        _mm_body,
        grid=(25,),
        in_specs=[pl.BlockSpec((400, _D), lambda i: (i, 0)),
                  pl.BlockSpec((_D, _D), lambda i: (0, 0))],
        out_specs=pl.BlockSpec((400, _D), lambda i: (i, 0)),
        out_shape=jax.ShapeDtypeStruct((_N, _D), jnp.float32),
    )(X, Wlin)

def kernel(X, vertex, edges, Wlin, degE, degV, W):
    Xp = _linear(X, Wlin)
    Xve = jnp.take(Xp, vertex, axis=0)
    Xe = jax.ops.segment_sum(Xve, edges, num_segments=_M)
    Xe = Xe * degE * W
    Xev = jnp.take(Xe, edges, axis=0)
    Xv = jax.ops.segment_sum(Xev, vertex, num_segments=_N)
    return Xv * degV
